# baseline (speedup 1.0000x reference)
import functools

import jax
import jax.numpy as jnp
from jax.experimental import pallas as pl
from jax.experimental.pallas import tpu as pltpu

N = 10000
E = 160000
D = 256
M = 256
OUT = 64
KAPPA = 0.99
REG_COEF = 0.001
FW_ITERS = 10
POW_ITERS = 30

BN = 1000
H = M // 2

_f32 = jnp.float32


def _l1_row_proj(v, k):
    absv = jnp.abs(v)
    u, _ = jax.lax.top_k(absv, absv.shape[0])
    css = jnp.cumsum(u)
    idx = jnp.arange(1, v.shape[0] + 1, dtype=v.dtype)
    cond = u - (css - k) / idx > 0
    rho = jnp.sum(cond).astype(jnp.int32)
    theta = (jnp.take(css, rho - 1) - k) / rho.astype(v.dtype)
    w = jnp.sign(v) * jnp.maximum(absv - theta, 0.0)
    return jnp.where(jnp.sum(absv) <= k, v, w)


def _init_body(x_ref, om1t_ref, ir_ref, bt_ref, z0_ref, z1_ref):
    bt = jnp.dot(x_ref[...], om1t_ref[...], preferred_element_type=_f32)
    bt_ref[...] = bt
    z = ir_ref[...] * jnp.maximum(bt, 0.0)
    z0_ref[...] = z[:, :H]
    z1_ref[...] = z[:, H:]


def _tc_init(x, om1t, ir):
    return pl.pallas_call(
        _init_body,
        grid=(N // BN,),
        in_specs=[
            pl.BlockSpec((BN, D), lambda i: (i, 0)),
            pl.BlockSpec((D, M), lambda i: (0, 0)),
            pl.BlockSpec((BN, 1), lambda i: (i, 0)),
        ],
        out_specs=[
            pl.BlockSpec((BN, M), lambda i: (i, 0)),
            pl.BlockSpec((BN, H), lambda i: (i, 0)),
            pl.BlockSpec((BN, H), lambda i: (i, 0)),
        ],
        out_shape=[
            jax.ShapeDtypeStruct((N, M), _f32),
            jax.ShapeDtypeStruct((N, H), _f32),
            jax.ShapeDtypeStruct((N, H), _f32),
        ],
    )(x, om1t, ir)


def _mid_body(xa0_ref, xa1_ref, ic_ref, ir_ref, wpt_ref, bt_ref, z0_ref, z1_ref):
    xa = jnp.concatenate([xa0_ref[...], xa1_ref[...]], axis=1)
    h = jnp.dot(xa, wpt_ref[...], preferred_element_type=_f32)
    xt = jnp.maximum(ic_ref[...] * h + bt_ref[...], 0.0)
    z = ir_ref[...] * xt
    z0_ref[...] = z[:, :H]
    z1_ref[...] = z[:, H:]


def _tc_mid(xa0, xa1, ic, ir, wpt, bt):
    return pl.pallas_call(
        _mid_body,
        grid=(N // BN,),
        in_specs=[
            pl.BlockSpec((BN, H), lambda i: (i, 0)),
            pl.BlockSpec((BN, H), lambda i: (i, 0)),
            pl.BlockSpec((BN, 1), lambda i: (i, 0)),
            pl.BlockSpec((BN, 1), lambda i: (i, 0)),
            pl.BlockSpec((M, M), lambda i: (0, 0)),
            pl.BlockSpec((BN, M), lambda i: (i, 0)),
        ],
        out_specs=[
            pl.BlockSpec((BN, H), lambda i: (i, 0)),
            pl.BlockSpec((BN, H), lambda i: (i, 0)),
        ],
        out_shape=[
            jax.ShapeDtypeStruct((N, H), _f32),
            jax.ShapeDtypeStruct((N, H), _f32),
        ],
    )(xa0, xa1, ic, ir, wpt, bt)


def _final_body(xa0_ref, xa1_ref, ic_ref, wpt_ref, bt_ref, hwt_ref, hb_ref, out_ref):
    xa = jnp.concatenate([xa0_ref[...], xa1_ref[...]], axis=1)
    h = jnp.dot(xa, wpt_ref[...], preferred_element_type=_f32)
    xt = jnp.maximum(ic_ref[...] * h + bt_ref[...], 0.0)
    out_ref[...] = jnp.dot(xt, hwt_ref[...], preferred_element_type=_f32) + hb_ref[...]


def _tc_final(xa0, xa1, ic, wpt, bt, hwt, hb):
    return pl.pallas_call(
        _final_body,
        grid=(N // BN,),
        in_specs=[
            pl.BlockSpec((BN, H), lambda i: (i, 0)),
            pl.BlockSpec((BN, H), lambda i: (i, 0)),
            pl.BlockSpec((BN, 1), lambda i: (i, 0)),
            pl.BlockSpec((M, M), lambda i: (0, 0)),
            pl.BlockSpec((BN, M), lambda i: (i, 0)),
            pl.BlockSpec((M, OUT), lambda i: (0, 0)),
            pl.BlockSpec((1, OUT), lambda i: (0, 0)),
        ],
        out_specs=pl.BlockSpec((BN, OUT), lambda i: (i, 0)),
        out_shape=jax.ShapeDtypeStruct((N, OUT), _f32),
    )(xa0, xa1, ic, wpt, bt, hwt, hb)


def _spmm(z0, z1, row, col):
    z = jnp.concatenate([z0, z1], axis=1)
    xa = jnp.zeros((N, M), dtype=_f32).at[col].add(z[row])
    return xa[:, :H], xa[:, H:]


def kernel(x, edge_index, W, Omega_1, head_w, head_b):
    row = edge_index[0]
    col = edge_index[1]
    ones = jnp.ones((E,), dtype=_f32)
    deg_r = jnp.zeros((N,), dtype=_f32).at[row].add(ones)
    deg_c = jnp.zeros((N,), dtype=_f32).at[col].add(ones)
    vals = 1.0 / (jnp.sqrt(jnp.maximum(deg_r[row], 1.0)) * jnp.sqrt(jnp.maximum(deg_c[col], 1.0)))

    v = jnp.ones((N,), dtype=_f32) / jnp.sqrt(jnp.float32(N))
    nrm = jnp.float32(1.0)
    for _ in range(POW_ITERS):
        w_ = jnp.zeros((N,), dtype=_f32).at[col].add(vals * v[row])
        nrm = jnp.linalg.norm(w_) + 1e-12
        v = w_ / nrm
    A_rho = jax.lax.stop_gradient(jnp.maximum(nrm, 1e-6))
    k = KAPPA / A_rho
    Wp = jax.vmap(lambda r: _l1_row_proj(r, k))(W)
    reg_loss = REG_COEF * jnp.sum(jax.nn.relu(jnp.sum(jnp.abs(Wp), axis=1) * A_rho - KAPPA))

    ir = (1.0 / jnp.sqrt(jnp.maximum(deg_r, 1.0)))[:, None]
    ic = (1.0 / jnp.sqrt(jnp.maximum(deg_c, 1.0)))[:, None]

    om1t = Omega_1.T
    wpt = Wp.T
    hwt = head_w.T
    hb = head_b.reshape(1, OUT)

    bt, z0, z1 = _tc_init(x, om1t, ir)
    for _ in range(FW_ITERS - 2):
        xa0, xa1 = _spmm(z0, z1, row, col)
        z0, z1 = _tc_mid(xa0, xa1, ic, ir, wpt, bt)
    xa0, xa1 = _spmm(z0, z1, row, col)
    logits = _tc_final(xa0, xa1, ic, wpt, bt, hwt, hb)
    return (logits, reg_loss)


# speedup vs baseline: 2.5725x; 2.5725x over previous
import functools

import jax
import jax.numpy as jnp
from jax.experimental import pallas as pl
from jax.experimental.pallas import tpu as pltpu
from jax.experimental.pallas import tpu_sc as plsc

N = 10000
E = 160000
D = 256
M = 256
OUT = 64
KAPPA = 0.99
REG_COEF = 0.001
FW_ITERS = 10
POW_ITERS = 30

BN = 1000
H = M // 2

_f32 = jnp.float32


def _l1_row_proj(v, k):
    absv = jnp.abs(v)
    u, _ = jax.lax.top_k(absv, absv.shape[0])
    css = jnp.cumsum(u)
    idx = jnp.arange(1, v.shape[0] + 1, dtype=v.dtype)
    cond = u - (css - k) / idx > 0
    rho = jnp.sum(cond).astype(jnp.int32)
    theta = (jnp.take(css, rho - 1) - k) / rho.astype(v.dtype)
    w = jnp.sign(v) * jnp.maximum(absv - theta, 0.0)
    return jnp.where(jnp.sum(absv) <= k, v, w)


_NC = 2
_NS = 16
_NW = _NC * _NS
_EPT = E // _NW
_GCHUNKS = (_EPT + 15) // 16
_EPT_PAD = _GCHUNKS * 16


def _gather_body(src_hbm, idx_hbm, out_hbm, src_v, idx_v, out_v):
    c = jax.lax.axis_index("c")
    s = jax.lax.axis_index("s")
    wid = s * _NC + c
    base = wid * _EPT
    idx_v[pl.ds(_EPT_PAD - 16, 16)] = jnp.zeros((16,), jnp.int32)
    pltpu.sync_copy(src_hbm, src_v)
    pltpu.sync_copy(idx_hbm.at[pl.ds(base, _EPT)], idx_v.at[pl.ds(0, _EPT)])

    def body(i, carry):
        iv = idx_v[pl.ds(i * 16, 16)]
        out_v[pl.ds(i * 16, 16)] = plsc.load_gather(src_v, [iv])
        return carry

    jax.lax.fori_loop(0, _GCHUNKS, body, 0)
    pltpu.sync_copy(out_v.at[pl.ds(0, _EPT)], out_hbm.at[pl.ds(base, _EPT)])


_sc_gather = pl.kernel(
    _gather_body,
    out_type=jax.ShapeDtypeStruct((E,), _f32),
    mesh=plsc.VectorSubcoreMesh(core_axis_name="c", subcore_axis_name="s"),
    scratch_types=[
        pltpu.VMEM((N,), _f32),
        pltpu.VMEM((_EPT_PAD,), jnp.int32),
        pltpu.VMEM((_EPT_PAD,), _f32),
    ],
    compiler_params=pltpu.CompilerParams(needs_layout_passes=False),
)


def _init_body(x_ref, om1t_ref, ir_ref, bt_ref, z0_ref, z1_ref):
    bt = jnp.dot(x_ref[...], om1t_ref[...], preferred_element_type=_f32)
    bt_ref[...] = bt
    z = ir_ref[...] * jnp.maximum(bt, 0.0)
    z0_ref[...] = z[:, :H]
    z1_ref[...] = z[:, H:]


def _tc_init(x, om1t, ir):
    return pl.pallas_call(
        _init_body,
        grid=(N // BN,),
        in_specs=[
            pl.BlockSpec((BN, D), lambda i: (i, 0)),
            pl.BlockSpec((D, M), lambda i: (0, 0)),
            pl.BlockSpec((BN, 1), lambda i: (i, 0)),
        ],
        out_specs=[
            pl.BlockSpec((BN, M), lambda i: (i, 0)),
            pl.BlockSpec((BN, H), lambda i: (i, 0)),
            pl.BlockSpec((BN, H), lambda i: (i, 0)),
        ],
        out_shape=[
            jax.ShapeDtypeStruct((N, M), _f32),
            jax.ShapeDtypeStruct((N, H), _f32),
            jax.ShapeDtypeStruct((N, H), _f32),
        ],
    )(x, om1t, ir)


def _mid_body(xa0_ref, xa1_ref, ic_ref, ir_ref, wpt_ref, bt_ref, z0_ref, z1_ref):
    xa = jnp.concatenate([xa0_ref[...], xa1_ref[...]], axis=1)
    h = jnp.dot(xa, wpt_ref[...], preferred_element_type=_f32)
    xt = jnp.maximum(ic_ref[...] * h + bt_ref[...], 0.0)
    z = ir_ref[...] * xt
    z0_ref[...] = z[:, :H]
    z1_ref[...] = z[:, H:]


def _tc_mid(xa0, xa1, ic, ir, wpt, bt):
    return pl.pallas_call(
        _mid_body,
        grid=(N // BN,),
        in_specs=[
            pl.BlockSpec((BN, H), lambda i: (i, 0)),
            pl.BlockSpec((BN, H), lambda i: (i, 0)),
            pl.BlockSpec((BN, 1), lambda i: (i, 0)),
            pl.BlockSpec((BN, 1), lambda i: (i, 0)),
            pl.BlockSpec((M, M), lambda i: (0, 0)),
            pl.BlockSpec((BN, M), lambda i: (i, 0)),
        ],
        out_specs=[
            pl.BlockSpec((BN, H), lambda i: (i, 0)),
            pl.BlockSpec((BN, H), lambda i: (i, 0)),
        ],
        out_shape=[
            jax.ShapeDtypeStruct((N, H), _f32),
            jax.ShapeDtypeStruct((N, H), _f32),
        ],
    )(xa0, xa1, ic, ir, wpt, bt)


def _final_body(xa0_ref, xa1_ref, ic_ref, wpt_ref, bt_ref, hwt_ref, hb_ref, out_ref):
    xa = jnp.concatenate([xa0_ref[...], xa1_ref[...]], axis=1)
    h = jnp.dot(xa, wpt_ref[...], preferred_element_type=_f32)
    xt = jnp.maximum(ic_ref[...] * h + bt_ref[...], 0.0)
    out_ref[...] = jnp.dot(xt, hwt_ref[...], preferred_element_type=_f32) + hb_ref[...]


def _tc_final(xa0, xa1, ic, wpt, bt, hwt, hb):
    return pl.pallas_call(
        _final_body,
        grid=(N // BN,),
        in_specs=[
            pl.BlockSpec((BN, H), lambda i: (i, 0)),
            pl.BlockSpec((BN, H), lambda i: (i, 0)),
            pl.BlockSpec((BN, 1), lambda i: (i, 0)),
            pl.BlockSpec((M, M), lambda i: (0, 0)),
            pl.BlockSpec((BN, M), lambda i: (i, 0)),
            pl.BlockSpec((M, OUT), lambda i: (0, 0)),
            pl.BlockSpec((1, OUT), lambda i: (0, 0)),
        ],
        out_specs=pl.BlockSpec((BN, OUT), lambda i: (i, 0)),
        out_shape=jax.ShapeDtypeStruct((N, OUT), _f32),
    )(xa0, xa1, ic, wpt, bt, hwt, hb)


def _spmm(z0, z1, row, col):
    z = jnp.concatenate([z0, z1], axis=1)
    xa = jnp.zeros((N, M), dtype=_f32).at[col].add(z[row])
    return xa[:, :H], xa[:, H:]


def kernel(x, edge_index, W, Omega_1, head_w, head_b):
    row = edge_index[0]
    col = edge_index[1]
    ones = jnp.ones((E,), dtype=_f32)
    deg_r = jnp.zeros((N,), dtype=_f32).at[row].add(ones)
    deg_c = jnp.zeros((N,), dtype=_f32).at[col].add(ones)
    vals = 1.0 / (jnp.sqrt(jnp.maximum(_sc_gather(deg_r, row), 1.0))
                  * jnp.sqrt(jnp.maximum(_sc_gather(deg_c, col), 1.0)))

    v = jnp.ones((N,), dtype=_f32) / jnp.sqrt(jnp.float32(N))
    nrm = jnp.float32(1.0)
    for _ in range(POW_ITERS):
        w_ = jnp.zeros((N,), dtype=_f32).at[col].add(vals * _sc_gather(v, row))
        nrm = jnp.linalg.norm(w_) + 1e-12
        v = w_ / nrm
    A_rho = jax.lax.stop_gradient(jnp.maximum(nrm, 1e-6))
    k = KAPPA / A_rho
    Wp = jax.vmap(lambda r: _l1_row_proj(r, k))(W)
    reg_loss = REG_COEF * jnp.sum(jax.nn.relu(jnp.sum(jnp.abs(Wp), axis=1) * A_rho - KAPPA))

    ir = (1.0 / jnp.sqrt(jnp.maximum(deg_r, 1.0)))[:, None]
    ic = (1.0 / jnp.sqrt(jnp.maximum(deg_c, 1.0)))[:, None]

    om1t = Omega_1.T
    wpt = Wp.T
    hwt = head_w.T
    hb = head_b.reshape(1, OUT)

    bt, z0, z1 = _tc_init(x, om1t, ir)
    for _ in range(FW_ITERS - 2):
        xa0, xa1 = _spmm(z0, z1, row, col)
        z0, z1 = _tc_mid(xa0, xa1, ic, ir, wpt, bt)
    xa0, xa1 = _spmm(z0, z1, row, col)
    logits = _tc_final(xa0, xa1, ic, wpt, bt, hwt, hb)
    return (logits, reg_loss)


# speedup vs baseline: 4.6048x; 1.7900x over previous
import functools

import jax
import jax.numpy as jnp
from jax.experimental import pallas as pl
from jax.experimental.pallas import tpu as pltpu
from jax.experimental.pallas import tpu_sc as plsc

N = 10000
E = 160000
D = 256
M = 256
OUT = 64
KAPPA = 0.99
REG_COEF = 0.001
FW_ITERS = 10
POW_ITERS = 30

BN = 1000
H = M // 2

_f32 = jnp.float32


def _l1_row_proj(v, k):
    absv = jnp.abs(v)
    u, _ = jax.lax.top_k(absv, absv.shape[0])
    css = jnp.cumsum(u)
    idx = jnp.arange(1, v.shape[0] + 1, dtype=v.dtype)
    cond = u - (css - k) / idx > 0
    rho = jnp.sum(cond).astype(jnp.int32)
    theta = (jnp.take(css, rho - 1) - k) / rho.astype(v.dtype)
    w = jnp.sign(v) * jnp.maximum(absv - theta, 0.0)
    return jnp.where(jnp.sum(absv) <= k, v, w)


_NC = 2
_NS = 16
_NW = _NC * _NS
_EPT = E // _NW
_GCHUNKS = (_EPT + 15) // 16
_EPT_PAD = _GCHUNKS * 16


def _gather_body(src_hbm, idx_hbm, out_hbm, src_v, idx_v, out_v):
    c = jax.lax.axis_index("c")
    s = jax.lax.axis_index("s")
    wid = s * _NC + c
    base = wid * _EPT
    idx_v[pl.ds(_EPT_PAD - 16, 16)] = jnp.zeros((16,), jnp.int32)
    pltpu.sync_copy(src_hbm, src_v)
    pltpu.sync_copy(idx_hbm.at[pl.ds(base, _EPT)], idx_v.at[pl.ds(0, _EPT)])

    def body(i, carry):
        iv = idx_v[pl.ds(i * 16, 16)]
        out_v[pl.ds(i * 16, 16)] = plsc.load_gather(src_v, [iv])
        return carry

    jax.lax.fori_loop(0, _GCHUNKS, body, 0)
    pltpu.sync_copy(out_v.at[pl.ds(0, _EPT)], out_hbm.at[pl.ds(base, _EPT)])


_sc_gather = pl.kernel(
    _gather_body,
    out_type=jax.ShapeDtypeStruct((E,), _f32),
    mesh=plsc.VectorSubcoreMesh(core_axis_name="c", subcore_axis_name="s"),
    scratch_types=[
        pltpu.VMEM((N,), _f32),
        pltpu.VMEM((_EPT_PAD,), jnp.int32),
        pltpu.VMEM((_EPT_PAD,), _f32),
    ],
    compiler_params=pltpu.CompilerParams(needs_layout_passes=False),
)


_EPS = E // _NS
_CB = 128
_NCHUNK = (_EPS + _CB - 1) // _CB
_EPS_PAD = _NCHUNK * _CB
_NACC = 10112
_RPT = _NACC // _NS
_RC_SHAPE = (_NS, _NCHUNK, _CB)


def _spmm_body(z_hbm, rowp_hbm, colp_hbm, xa_hbm,
               idxr_v, idxc_v, rowbuf_v, sem0, acc_sh):
    c = jax.lax.axis_index("c")
    s = jax.lax.axis_index("s")

    pltpu.sync_copy(rowp_hbm.at[c, s], idxr_v)
    pltpu.sync_copy(colp_hbm.at[s], idxc_v)

    zv = jnp.zeros((16,), dtype=_f32)
    def zbody(i, carry):
        for l in range(H // 16):
            rowbuf_v[i, pl.ds(l * 16, 16)] = zv
        return carry
    jax.lax.fori_loop(0, _CB, zbody, 0)
    zbase = s * _RPT
    for q in range(4):
        pltpu.sync_copy(rowbuf_v, acc_sh.at[pl.ds(zbase + q * _CB, _CB)])
    pltpu.sync_copy(rowbuf_v.at[pl.ds(0, _RPT - 4 * _CB)],
                    acc_sh.at[pl.ds(zbase + 4 * _CB, _RPT - 4 * _CB)])
    plsc.subcore_barrier()

    def lbody(j, carry):
        pltpu.async_copy(z_hbm.at[idxr_v.at[j]], rowbuf_v, sem0).wait()
        pltpu.sync_copy(rowbuf_v, acc_sh.at[idxc_v.at[j]], add=True)
        return carry

    jax.lax.fori_loop(0, _NCHUNK, lbody, 0)
    plsc.subcore_barrier()

    @pl.when(s < 2)
    def _():
        off = s * 632
        pltpu.sync_copy(acc_sh.at[pl.ds(off, 632)],
                        xa_hbm.at[pl.ds(c * N + off, 632)])

    @pl.when(s >= 2)
    def _():
        off = 1264 + (s - 2) * 624
        pltpu.sync_copy(acc_sh.at[pl.ds(off, 624)],
                        xa_hbm.at[pl.ds(c * N + off, 624)])


_sc_spmm_call = pl.kernel(
    _spmm_body,
    out_type=jax.ShapeDtypeStruct((2 * N, H), _f32),
    mesh=plsc.VectorSubcoreMesh(core_axis_name="c", subcore_axis_name="s"),
    scratch_types=[
        pltpu.VMEM((_NCHUNK, _CB), jnp.int32),
        pltpu.VMEM((_NCHUNK, _CB), jnp.int32),
        pltpu.VMEM((_CB, H), _f32),
        pltpu.SemaphoreType.DMA,
        pltpu.VMEM_SHARED((_NACC, H), _f32),
    ],
    compiler_params=pltpu.CompilerParams(needs_layout_passes=False),
)


def _prep_edge_chunks(row, col):
    npad = _EPS_PAD - _EPS
    rpad = (jnp.arange(npad, dtype=jnp.int32) * 64) % N
    cpad = N + jnp.arange(npad, dtype=jnp.int32) % (_NACC - N)
    r2 = jnp.concatenate(
        [row.reshape(_NS, _EPS), jnp.tile(rpad, (_NS, 1))], axis=1
    ).reshape(_RC_SHAPE)
    c2 = jnp.concatenate(
        [col.reshape(_NS, _EPS), jnp.tile(cpad, (_NS, 1))], axis=1
    ).reshape(_RC_SHAPE)
    rowp = jnp.stack([r2, r2 + N], axis=0)
    return rowp, c2


def _init_body(x_ref, om1t_ref, ir_ref, bt_ref, z_ref):
    bt = jnp.dot(x_ref[...], om1t_ref[...], preferred_element_type=_f32)
    bt_ref[...] = bt
    z = ir_ref[...] * jnp.maximum(bt, 0.0)
    z_ref[...] = jnp.stack([z[:, :H], z[:, H:]], axis=0)


def _tc_init(x, om1t, ir):
    return pl.pallas_call(
        _init_body,
        grid=(N // BN,),
        in_specs=[
            pl.BlockSpec((BN, D), lambda i: (i, 0)),
            pl.BlockSpec((D, M), lambda i: (0, 0)),
            pl.BlockSpec((BN, 1), lambda i: (i, 0)),
        ],
        out_specs=[
            pl.BlockSpec((BN, M), lambda i: (i, 0)),
            pl.BlockSpec((2, BN, H), lambda i: (0, i, 0)),
        ],
        out_shape=[
            jax.ShapeDtypeStruct((N, M), _f32),
            jax.ShapeDtypeStruct((2, N, H), _f32),
        ],
    )(x, om1t, ir)


def _mid_body(xa0_ref, xa1_ref, ic_ref, ir_ref, wpt_ref, bt_ref, z_ref):
    xa = jnp.concatenate([xa0_ref[...], xa1_ref[...]], axis=1)
    h = jnp.dot(xa, wpt_ref[...], preferred_element_type=_f32)
    xt = jnp.maximum(ic_ref[...] * h + bt_ref[...], 0.0)
    z = ir_ref[...] * xt
    z_ref[...] = jnp.stack([z[:, :H], z[:, H:]], axis=0)


def _tc_mid(xa, ic, ir, wpt, bt):
    nb = N // BN
    return pl.pallas_call(
        _mid_body,
        grid=(nb,),
        in_specs=[
            pl.BlockSpec((BN, H), lambda i: (i, 0)),
            pl.BlockSpec((BN, H), lambda i: (i + N // BN, 0)),
            pl.BlockSpec((BN, 1), lambda i: (i, 0)),
            pl.BlockSpec((BN, 1), lambda i: (i, 0)),
            pl.BlockSpec((M, M), lambda i: (0, 0)),
            pl.BlockSpec((BN, M), lambda i: (i, 0)),
        ],
        out_specs=pl.BlockSpec((2, BN, H), lambda i: (0, i, 0)),
        out_shape=jax.ShapeDtypeStruct((2, N, H), _f32),
    )(xa, xa, ic, ir, wpt, bt)


def _final_body(xa0_ref, xa1_ref, ic_ref, wpt_ref, bt_ref, hwt_ref, hb_ref, out_ref):
    xa = jnp.concatenate([xa0_ref[...], xa1_ref[...]], axis=1)
    h = jnp.dot(xa, wpt_ref[...], preferred_element_type=_f32)
    xt = jnp.maximum(ic_ref[...] * h + bt_ref[...], 0.0)
    out_ref[...] = jnp.dot(xt, hwt_ref[...], preferred_element_type=_f32) + hb_ref[...]


def _tc_final(xa, ic, wpt, bt, hwt, hb):
    return pl.pallas_call(
        _final_body,
        grid=(N // BN,),
        in_specs=[
            pl.BlockSpec((BN, H), lambda i: (i, 0)),
            pl.BlockSpec((BN, H), lambda i: (i + N // BN, 0)),
            pl.BlockSpec((BN, 1), lambda i: (i, 0)),
            pl.BlockSpec((M, M), lambda i: (0, 0)),
            pl.BlockSpec((BN, M), lambda i: (i, 0)),
            pl.BlockSpec((M, OUT), lambda i: (0, 0)),
            pl.BlockSpec((1, OUT), lambda i: (0, 0)),
        ],
        out_specs=pl.BlockSpec((BN, OUT), lambda i: (i, 0)),
        out_shape=jax.ShapeDtypeStruct((N, OUT), _f32),
    )(xa, xa, ic, wpt, bt, hwt, hb)


def kernel(x, edge_index, W, Omega_1, head_w, head_b):
    row = edge_index[0]
    col = edge_index[1]
    ones = jnp.ones((E,), dtype=_f32)
    deg_r = jnp.zeros((N,), dtype=_f32).at[row].add(ones)
    deg_c = jnp.zeros((N,), dtype=_f32).at[col].add(ones)
    vals = 1.0 / (jnp.sqrt(jnp.maximum(_sc_gather(deg_r, row), 1.0))
                  * jnp.sqrt(jnp.maximum(_sc_gather(deg_c, col), 1.0)))

    v = jnp.ones((N,), dtype=_f32) / jnp.sqrt(jnp.float32(N))
    nrm = jnp.float32(1.0)
    for _ in range(POW_ITERS):
        w_ = jnp.zeros((N,), dtype=_f32).at[col].add(vals * _sc_gather(v, row))
        nrm = jnp.linalg.norm(w_) + 1e-12
        v = w_ / nrm
    A_rho = jax.lax.stop_gradient(jnp.maximum(nrm, 1e-6))
    k = KAPPA / A_rho
    Wp = jax.vmap(lambda r: _l1_row_proj(r, k))(W)
    reg_loss = REG_COEF * jnp.sum(jax.nn.relu(jnp.sum(jnp.abs(Wp), axis=1) * A_rho - KAPPA))

    ir = (1.0 / jnp.sqrt(jnp.maximum(deg_r, 1.0)))[:, None]
    ic = (1.0 / jnp.sqrt(jnp.maximum(deg_c, 1.0)))[:, None]

    om1t = Omega_1.T
    wpt = Wp.T
    hwt = head_w.T
    hb = head_b.reshape(1, OUT)

    rowp, colp = _prep_edge_chunks(row, col)
    bt, z3 = _tc_init(x, om1t, ir)
    for _ in range(FW_ITERS - 2):
        xa = _sc_spmm_call(z3.reshape(2 * N, H), rowp, colp)
        z3 = _tc_mid(xa, ic, ir, wpt, bt)
    xa = _sc_spmm_call(z3.reshape(2 * N, H), rowp, colp)
    logits = _tc_final(xa, ic, wpt, bt, hwt, hb)
    return (logits, reg_loss)


# speedup vs baseline: 4.6105x; 1.0013x over previous
import functools

import jax
import jax.numpy as jnp
from jax.experimental import pallas as pl
from jax.experimental.pallas import tpu as pltpu
from jax.experimental.pallas import tpu_sc as plsc

N = 10000
E = 160000
D = 256
M = 256
OUT = 64
KAPPA = 0.99
REG_COEF = 0.001
FW_ITERS = 10
POW_ITERS = 30

BN = 1000
H = M // 2

_f32 = jnp.float32


def _sort_body(wt_ref, out_ref):
    a = jnp.abs(wt_ref[...])
    n = a.shape[0]
    i = jax.lax.broadcasted_iota(jnp.int32, (n, 1), 0)
    k = 2
    while k <= n:
        j = k // 2
        while j >= 1:
            r = a.reshape(n // (2 * j), 2, j, n)
            p = jnp.concatenate([r[:, 1:2], r[:, 0:1]], axis=1).reshape(n, n)
            up = (i & k) != 0
            take_min = jnp.logical_xor(up, (i & j) != 0)
            a = jnp.where(take_min, jnp.minimum(a, p), jnp.maximum(a, p))
            j //= 2
        k *= 2
    out_ref[...] = a


def _tc_sortdesc(wt):
    n = wt.shape[0]
    return pl.pallas_call(
        _sort_body,
        out_shape=jax.ShapeDtypeStruct((n, n), _f32),
    )(wt)


def _l1_row_proj(v, u, k):
    absv = jnp.abs(v)
    css = jnp.cumsum(u)
    idx = jnp.arange(1, v.shape[0] + 1, dtype=v.dtype)
    cond = u - (css - k) / idx > 0
    rho = jnp.sum(cond).astype(jnp.int32)
    theta = (jnp.take(css, rho - 1) - k) / rho.astype(v.dtype)
    w = jnp.sign(v) * jnp.maximum(absv - theta, 0.0)
    return jnp.where(jnp.sum(absv) <= k, v, w)


_NC = 2
_NS = 16
_NW = _NC * _NS
_EPT = E // _NW
_GCHUNKS = (_EPT + 15) // 16
_EPT_PAD = _GCHUNKS * 16


def _gather_body(src_hbm, idx_hbm, out_hbm, src_v, idx_v, out_v):
    c = jax.lax.axis_index("c")
    s = jax.lax.axis_index("s")
    wid = s * _NC + c
    base = wid * _EPT
    idx_v[pl.ds(_EPT_PAD - 16, 16)] = jnp.zeros((16,), jnp.int32)
    pltpu.sync_copy(src_hbm, src_v)
    pltpu.sync_copy(idx_hbm.at[pl.ds(base, _EPT)], idx_v.at[pl.ds(0, _EPT)])

    def body(i, carry):
        iv = idx_v[pl.ds(i * 16, 16)]
        out_v[pl.ds(i * 16, 16)] = plsc.load_gather(src_v, [iv])
        return carry

    jax.lax.fori_loop(0, _GCHUNKS, body, 0)
    pltpu.sync_copy(out_v.at[pl.ds(0, _EPT)], out_hbm.at[pl.ds(base, _EPT)])


_sc_gather = pl.kernel(
    _gather_body,
    out_type=jax.ShapeDtypeStruct((E,), _f32),
    mesh=plsc.VectorSubcoreMesh(core_axis_name="c", subcore_axis_name="s"),
    scratch_types=[
        pltpu.VMEM((N,), _f32),
        pltpu.VMEM((_EPT_PAD,), jnp.int32),
        pltpu.VMEM((_EPT_PAD,), _f32),
    ],
    compiler_params=pltpu.CompilerParams(needs_layout_passes=False),
)


_EPS = E // _NS
_CB = 128
_NCHUNK = (_EPS + _CB - 1) // _CB
_EPS_PAD = _NCHUNK * _CB
_NACC = 10112
_RPT = _NACC // _NS
_RC_SHAPE = (_NS, _NCHUNK, _CB)


def _spmm_body(z_hbm, rowp_hbm, colp_hbm, xa_hbm,
               idxr_v, idxc_v, rowbuf_v, sem0, acc_sh):
    c = jax.lax.axis_index("c")
    s = jax.lax.axis_index("s")

    pltpu.sync_copy(rowp_hbm.at[c, s], idxr_v)
    pltpu.sync_copy(colp_hbm.at[s], idxc_v)

    zv = jnp.zeros((16,), dtype=_f32)
    def zbody(i, carry):
        for l in range(H // 16):
            rowbuf_v[i, pl.ds(l * 16, 16)] = zv
        return carry
    jax.lax.fori_loop(0, _CB, zbody, 0)
    zbase = s * _RPT
    for q in range(4):
        pltpu.sync_copy(rowbuf_v, acc_sh.at[pl.ds(zbase + q * _CB, _CB)])
    pltpu.sync_copy(rowbuf_v.at[pl.ds(0, _RPT - 4 * _CB)],
                    acc_sh.at[pl.ds(zbase + 4 * _CB, _RPT - 4 * _CB)])
    plsc.subcore_barrier()

    def lbody(j, carry):
        pltpu.async_copy(z_hbm.at[idxr_v.at[j]], rowbuf_v, sem0).wait()
        pltpu.sync_copy(rowbuf_v, acc_sh.at[idxc_v.at[j]], add=True)
        return carry

    jax.lax.fori_loop(0, _NCHUNK, lbody, 0)
    plsc.subcore_barrier()

    @pl.when(s < 2)
    def _():
        off = s * 632
        pltpu.sync_copy(acc_sh.at[pl.ds(off, 632)],
                        xa_hbm.at[pl.ds(c * N + off, 632)])

    @pl.when(s >= 2)
    def _():
        off = 1264 + (s - 2) * 624
        pltpu.sync_copy(acc_sh.at[pl.ds(off, 624)],
                        xa_hbm.at[pl.ds(c * N + off, 624)])


_sc_spmm_call = pl.kernel(
    _spmm_body,
    out_type=jax.ShapeDtypeStruct((2 * N, H), _f32),
    mesh=plsc.VectorSubcoreMesh(core_axis_name="c", subcore_axis_name="s"),
    scratch_types=[
        pltpu.VMEM((_NCHUNK, _CB), jnp.int32),
        pltpu.VMEM((_NCHUNK, _CB), jnp.int32),
        pltpu.VMEM((_CB, H), _f32),
        pltpu.SemaphoreType.DMA,
        pltpu.VMEM_SHARED((_NACC, H), _f32),
    ],
    compiler_params=pltpu.CompilerParams(needs_layout_passes=False),
)


def _prep_edge_chunks(row, col):
    npad = _EPS_PAD - _EPS
    rpad = (jnp.arange(npad, dtype=jnp.int32) * 64) % N
    cpad = N + jnp.arange(npad, dtype=jnp.int32) % (_NACC - N)
    r2 = jnp.concatenate(
        [row.reshape(_NS, _EPS), jnp.tile(rpad, (_NS, 1))], axis=1
    ).reshape(_RC_SHAPE)
    c2 = jnp.concatenate(
        [col.reshape(_NS, _EPS), jnp.tile(cpad, (_NS, 1))], axis=1
    ).reshape(_RC_SHAPE)
    rowp = jnp.stack([r2, r2 + N], axis=0)
    return rowp, c2


def _init_body(x_ref, om1t_ref, ir_ref, bt_ref, z_ref):
    bt = jnp.dot(x_ref[...], om1t_ref[...], preferred_element_type=_f32)
    bt_ref[...] = bt
    z = ir_ref[...] * jnp.maximum(bt, 0.0)
    z_ref[...] = jnp.stack([z[:, :H], z[:, H:]], axis=0)


def _tc_init(x, om1t, ir):
    return pl.pallas_call(
        _init_body,
        grid=(N // BN,),
        in_specs=[
            pl.BlockSpec((BN, D), lambda i: (i, 0)),
            pl.BlockSpec((D, M), lambda i: (0, 0)),
            pl.BlockSpec((BN, 1), lambda i: (i, 0)),
        ],
        out_specs=[
            pl.BlockSpec((BN, M), lambda i: (i, 0)),
            pl.BlockSpec((2, BN, H), lambda i: (0, i, 0)),
        ],
        out_shape=[
            jax.ShapeDtypeStruct((N, M), _f32),
            jax.ShapeDtypeStruct((2, N, H), _f32),
        ],
    )(x, om1t, ir)


def _mid_body(xa0_ref, xa1_ref, ic_ref, ir_ref, wpt_ref, bt_ref, z_ref):
    xa = jnp.concatenate([xa0_ref[...], xa1_ref[...]], axis=1)
    h = jnp.dot(xa, wpt_ref[...], preferred_element_type=_f32)
    xt = jnp.maximum(ic_ref[...] * h + bt_ref[...], 0.0)
    z = ir_ref[...] * xt
    z_ref[...] = jnp.stack([z[:, :H], z[:, H:]], axis=0)


def _tc_mid(xa, ic, ir, wpt, bt):
    nb = N // BN
    return pl.pallas_call(
        _mid_body,
        grid=(nb,),
        in_specs=[
            pl.BlockSpec((BN, H), lambda i: (i, 0)),
            pl.BlockSpec((BN, H), lambda i: (i + N // BN, 0)),
            pl.BlockSpec((BN, 1), lambda i: (i, 0)),
            pl.BlockSpec((BN, 1), lambda i: (i, 0)),
            pl.BlockSpec((M, M), lambda i: (0, 0)),
            pl.BlockSpec((BN, M), lambda i: (i, 0)),
        ],
        out_specs=pl.BlockSpec((2, BN, H), lambda i: (0, i, 0)),
        out_shape=jax.ShapeDtypeStruct((2, N, H), _f32),
    )(xa, xa, ic, ir, wpt, bt)


def _final_body(xa0_ref, xa1_ref, ic_ref, wpt_ref, bt_ref, hwt_ref, hb_ref, out_ref):
    xa = jnp.concatenate([xa0_ref[...], xa1_ref[...]], axis=1)
    h = jnp.dot(xa, wpt_ref[...], preferred_element_type=_f32)
    xt = jnp.maximum(ic_ref[...] * h + bt_ref[...], 0.0)
    out_ref[...] = jnp.dot(xt, hwt_ref[...], preferred_element_type=_f32) + hb_ref[...]


def _tc_final(xa, ic, wpt, bt, hwt, hb):
    return pl.pallas_call(
        _final_body,
        grid=(N // BN,),
        in_specs=[
            pl.BlockSpec((BN, H), lambda i: (i, 0)),
            pl.BlockSpec((BN, H), lambda i: (i + N // BN, 0)),
            pl.BlockSpec((BN, 1), lambda i: (i, 0)),
            pl.BlockSpec((M, M), lambda i: (0, 0)),
            pl.BlockSpec((BN, M), lambda i: (i, 0)),
            pl.BlockSpec((M, OUT), lambda i: (0, 0)),
            pl.BlockSpec((1, OUT), lambda i: (0, 0)),
        ],
        out_specs=pl.BlockSpec((BN, OUT), lambda i: (i, 0)),
        out_shape=jax.ShapeDtypeStruct((N, OUT), _f32),
    )(xa, xa, ic, wpt, bt, hwt, hb)


def kernel(x, edge_index, W, Omega_1, head_w, head_b):
    row = edge_index[0]
    col = edge_index[1]
    ones = jnp.ones((E,), dtype=_f32)
    deg_r = jnp.zeros((N,), dtype=_f32).at[row].add(ones)
    deg_c = jnp.zeros((N,), dtype=_f32).at[col].add(ones)
    vals = 1.0 / (jnp.sqrt(jnp.maximum(_sc_gather(deg_r, row), 1.0))
                  * jnp.sqrt(jnp.maximum(_sc_gather(deg_c, col), 1.0)))

    v = jnp.ones((N,), dtype=_f32) / jnp.sqrt(jnp.float32(N))
    nrm = jnp.float32(1.0)
    for _ in range(POW_ITERS):
        w_ = jnp.zeros((N,), dtype=_f32).at[col].add(vals * _sc_gather(v, row))
        nrm = jnp.linalg.norm(w_) + 1e-12
        v = w_ / nrm
    A_rho = jax.lax.stop_gradient(jnp.maximum(nrm, 1e-6))
    k = KAPPA / A_rho
    u_all = _tc_sortdesc(W.T).T
    Wp = jax.vmap(lambda r, u: _l1_row_proj(r, u, k))(W, u_all)
    reg_loss = REG_COEF * jnp.sum(jax.nn.relu(jnp.sum(jnp.abs(Wp), axis=1) * A_rho - KAPPA))

    ir = (1.0 / jnp.sqrt(jnp.maximum(deg_r, 1.0)))[:, None]
    ic = (1.0 / jnp.sqrt(jnp.maximum(deg_c, 1.0)))[:, None]

    om1t = Omega_1.T
    wpt = Wp.T
    hwt = head_w.T
    hb = head_b.reshape(1, OUT)

    rowp, colp = _prep_edge_chunks(row, col)
    bt, z3 = _tc_init(x, om1t, ir)
    for _ in range(FW_ITERS - 2):
        xa = _sc_spmm_call(z3.reshape(2 * N, H), rowp, colp)
        z3 = _tc_mid(xa, ic, ir, wpt, bt)
    xa = _sc_spmm_call(z3.reshape(2 * N, H), rowp, colp)
    logits = _tc_final(xa, ic, wpt, bt, hwt, hb)
    return (logits, reg_loss)


# speedup vs baseline: 8.0554x; 1.7472x over previous
import functools

import jax
import jax.numpy as jnp
from jax.experimental import pallas as pl
from jax.experimental.pallas import tpu as pltpu
from jax.experimental.pallas import tpu_sc as plsc

N = 10000
E = 160000
D = 256
M = 256
OUT = 64
KAPPA = 0.99
REG_COEF = 0.001
FW_ITERS = 10
POW_ITERS = 30

BN = 1000
H = M // 2

_f32 = jnp.float32


def _sort_body(wt_ref, out_ref):
    a = jnp.abs(wt_ref[...])
    n = a.shape[0]
    i = jax.lax.broadcasted_iota(jnp.int32, (n, 1), 0)
    k = 2
    while k <= n:
        j = k // 2
        while j >= 1:
            r = a.reshape(n // (2 * j), 2, j, n)
            p = jnp.concatenate([r[:, 1:2], r[:, 0:1]], axis=1).reshape(n, n)
            up = (i & k) != 0
            take_min = jnp.logical_xor(up, (i & j) != 0)
            a = jnp.where(take_min, jnp.minimum(a, p), jnp.maximum(a, p))
            j //= 2
        k *= 2
    out_ref[...] = a


def _tc_sortdesc(wt):
    n = wt.shape[0]
    return pl.pallas_call(
        _sort_body,
        out_shape=jax.ShapeDtypeStruct((n, n), _f32),
    )(wt)


def _l1_row_proj(v, u, k):
    absv = jnp.abs(v)
    css = jnp.cumsum(u)
    idx = jnp.arange(1, v.shape[0] + 1, dtype=v.dtype)
    cond = u - (css - k) / idx > 0
    rho = jnp.sum(cond).astype(jnp.int32)
    theta = (jnp.take(css, rho - 1) - k) / rho.astype(v.dtype)
    w = jnp.sign(v) * jnp.maximum(absv - theta, 0.0)
    return jnp.where(jnp.sum(absv) <= k, v, w)


_NC = 2
_NS = 16
_NW = _NC * _NS
_EPT = E // _NW
_GCHUNKS = (_EPT + 15) // 16
_EPT_PAD = _GCHUNKS * 16


def _gather_body(src_hbm, idx_hbm, out_hbm, src_v, idx_v, out_v):
    c = jax.lax.axis_index("c")
    s = jax.lax.axis_index("s")
    wid = s * _NC + c
    base = wid * _EPT
    idx_v[pl.ds(_EPT_PAD - 16, 16)] = jnp.zeros((16,), jnp.int32)
    pltpu.sync_copy(src_hbm, src_v)
    pltpu.sync_copy(idx_hbm.at[pl.ds(base, _EPT)], idx_v.at[pl.ds(0, _EPT)])

    def body(i, carry):
        iv = idx_v[pl.ds(i * 16, 16)]
        out_v[pl.ds(i * 16, 16)] = plsc.load_gather(src_v, [iv])
        return carry

    jax.lax.fori_loop(0, _GCHUNKS, body, 0)
    pltpu.sync_copy(out_v.at[pl.ds(0, _EPT)], out_hbm.at[pl.ds(base, _EPT)])


_sc_gather = pl.kernel(
    _gather_body,
    out_type=jax.ShapeDtypeStruct((E,), _f32),
    mesh=plsc.VectorSubcoreMesh(core_axis_name="c", subcore_axis_name="s"),
    scratch_types=[
        pltpu.VMEM((N,), _f32),
        pltpu.VMEM((_EPT_PAD,), jnp.int32),
        pltpu.VMEM((_EPT_PAD,), _f32),
    ],
    compiler_params=pltpu.CompilerParams(needs_layout_passes=False),
)


_EPS = E // _NS
_CB = 128
_NCHUNK = (_EPS + _CB - 1) // _CB
_EPS_PAD = _NCHUNK * _CB
_NACC = 10112
_RPT = _NACC // _NS
_RC_SHAPE = (_NS, _NCHUNK, _CB)


def _spmm_body(z_hbm, rowp_hbm, colp_hbm, xa_hbm,
               idxr_v, idxc_v, rowbuf_v, sem0, acc_sh):
    c = jax.lax.axis_index("c")
    s = jax.lax.axis_index("s")

    pltpu.sync_copy(rowp_hbm.at[c, s], idxr_v)
    pltpu.sync_copy(colp_hbm.at[s], idxc_v)

    zv = jnp.zeros((16,), dtype=_f32)
    def zbody(i, carry):
        for l in range(H // 16):
            rowbuf_v[i, pl.ds(l * 16, 16)] = zv
        return carry
    jax.lax.fori_loop(0, _CB, zbody, 0)
    zbase = s * _RPT
    for q in range(4):
        pltpu.sync_copy(rowbuf_v, acc_sh.at[pl.ds(zbase + q * _CB, _CB)])
    pltpu.sync_copy(rowbuf_v.at[pl.ds(0, _RPT - 4 * _CB)],
                    acc_sh.at[pl.ds(zbase + 4 * _CB, _RPT - 4 * _CB)])
    plsc.subcore_barrier()

    def lbody(j, carry):
        pltpu.async_copy(z_hbm.at[idxr_v.at[j]], rowbuf_v, sem0).wait()
        pltpu.sync_copy(rowbuf_v, acc_sh.at[idxc_v.at[j]], add=True)
        return carry

    jax.lax.fori_loop(0, _NCHUNK, lbody, 0)
    plsc.subcore_barrier()

    @pl.when(s < 2)
    def _():
        off = s * 632
        pltpu.sync_copy(acc_sh.at[pl.ds(off, 632)],
                        xa_hbm.at[pl.ds(c * N + off, 632)])

    @pl.when(s >= 2)
    def _():
        off = 1264 + (s - 2) * 624
        pltpu.sync_copy(acc_sh.at[pl.ds(off, 624)],
                        xa_hbm.at[pl.ds(c * N + off, 624)])


_sc_spmm_call = pl.kernel(
    _spmm_body,
    out_type=jax.ShapeDtypeStruct((2 * N, H), _f32),
    mesh=plsc.VectorSubcoreMesh(core_axis_name="c", subcore_axis_name="s"),
    scratch_types=[
        pltpu.VMEM((_NCHUNK, _CB), jnp.int32),
        pltpu.VMEM((_NCHUNK, _CB), jnp.int32),
        pltpu.VMEM((_CB, H), _f32),
        pltpu.SemaphoreType.DMA,
        pltpu.VMEM_SHARED((_NACC, H), _f32),
    ],
    compiler_params=pltpu.CompilerParams(needs_layout_passes=False),
)


def _prep_edge_chunks(row, col):
    npad = _EPS_PAD - _EPS
    rpad = (jnp.arange(npad, dtype=jnp.int32) * 64) % N
    cpad = N + jnp.arange(npad, dtype=jnp.int32) % (_NACC - N)
    r2 = jnp.concatenate(
        [row.reshape(_NS, _EPS), jnp.tile(rpad, (_NS, 1))], axis=1
    ).reshape(_RC_SHAPE)
    c2 = jnp.concatenate(
        [col.reshape(_NS, _EPS), jnp.tile(cpad, (_NS, 1))], axis=1
    ).reshape(_RC_SHAPE)
    rowp = jnp.stack([r2, r2 + N], axis=0)
    return rowp, c2


def _init_body(x_ref, om1t_ref, ir_ref, bt_ref, z_ref):
    bt = jnp.dot(x_ref[...], om1t_ref[...], preferred_element_type=_f32)
    bt_ref[...] = bt
    z = ir_ref[...] * jnp.maximum(bt, 0.0)
    z_ref[...] = jnp.stack([z[:, :H], z[:, H:]], axis=0)


def _tc_init(x, om1t, ir):
    return pl.pallas_call(
        _init_body,
        grid=(N // BN,),
        in_specs=[
            pl.BlockSpec((BN, D), lambda i: (i, 0)),
            pl.BlockSpec((D, M), lambda i: (0, 0)),
            pl.BlockSpec((BN, 1), lambda i: (i, 0)),
        ],
        out_specs=[
            pl.BlockSpec((BN, M), lambda i: (i, 0)),
            pl.BlockSpec((2, BN, H), lambda i: (0, i, 0)),
        ],
        out_shape=[
            jax.ShapeDtypeStruct((N, M), _f32),
            jax.ShapeDtypeStruct((2, N, H), _f32),
        ],
    )(x, om1t, ir)


def _mid_body(xa0_ref, xa1_ref, ic_ref, ir_ref, wpt_ref, bt_ref, z_ref):
    xa = jnp.concatenate([xa0_ref[...], xa1_ref[...]], axis=1)
    h = jnp.dot(xa, wpt_ref[...], preferred_element_type=_f32)
    xt = jnp.maximum(ic_ref[...] * h + bt_ref[...], 0.0)
    z = ir_ref[...] * xt
    z_ref[...] = jnp.stack([z[:, :H], z[:, H:]], axis=0)


def _tc_mid(xa, ic, ir, wpt, bt):
    nb = N // BN
    return pl.pallas_call(
        _mid_body,
        grid=(nb,),
        in_specs=[
            pl.BlockSpec((BN, H), lambda i: (i, 0)),
            pl.BlockSpec((BN, H), lambda i: (i + N // BN, 0)),
            pl.BlockSpec((BN, 1), lambda i: (i, 0)),
            pl.BlockSpec((BN, 1), lambda i: (i, 0)),
            pl.BlockSpec((M, M), lambda i: (0, 0)),
            pl.BlockSpec((BN, M), lambda i: (i, 0)),
        ],
        out_specs=pl.BlockSpec((2, BN, H), lambda i: (0, i, 0)),
        out_shape=jax.ShapeDtypeStruct((2, N, H), _f32),
    )(xa, xa, ic, ir, wpt, bt)


def _final_body(xa0_ref, xa1_ref, ic_ref, wpt_ref, bt_ref, hwt_ref, hb_ref, out_ref):
    xa = jnp.concatenate([xa0_ref[...], xa1_ref[...]], axis=1)
    h = jnp.dot(xa, wpt_ref[...], preferred_element_type=_f32)
    xt = jnp.maximum(ic_ref[...] * h + bt_ref[...], 0.0)
    out_ref[...] = jnp.dot(xt, hwt_ref[...], preferred_element_type=_f32) + hb_ref[...]


def _tc_final(xa, ic, wpt, bt, hwt, hb):
    return pl.pallas_call(
        _final_body,
        grid=(N // BN,),
        in_specs=[
            pl.BlockSpec((BN, H), lambda i: (i, 0)),
            pl.BlockSpec((BN, H), lambda i: (i + N // BN, 0)),
            pl.BlockSpec((BN, 1), lambda i: (i, 0)),
            pl.BlockSpec((M, M), lambda i: (0, 0)),
            pl.BlockSpec((BN, M), lambda i: (i, 0)),
            pl.BlockSpec((M, OUT), lambda i: (0, 0)),
            pl.BlockSpec((1, OUT), lambda i: (0, 0)),
        ],
        out_specs=pl.BlockSpec((BN, OUT), lambda i: (i, 0)),
        out_shape=jax.ShapeDtypeStruct((N, OUT), _f32),
    )(xa, xa, ic, wpt, bt, hwt, hb)


def kernel(x, edge_index, W, Omega_1, head_w, head_b):
    row = edge_index[0]
    col = edge_index[1]
    ones = jnp.ones((E,), dtype=_f32)
    deg_r = jnp.zeros((N,), dtype=_f32).at[row].add(ones)
    deg_c = jnp.zeros((N,), dtype=_f32).at[col].add(ones)
    col_s, row_s = jax.lax.sort((col, row), num_keys=1, is_stable=False)
    vals_s = 1.0 / (jnp.sqrt(jnp.maximum(_sc_gather(deg_r, row_s), 1.0))
                    * jnp.sqrt(jnp.maximum(_sc_gather(deg_c, col_s), 1.0)))

    v = jnp.ones((N,), dtype=_f32) / jnp.sqrt(jnp.float32(N))
    nrm = jnp.float32(1.0)
    for _ in range(POW_ITERS):
        w_ = jnp.zeros((N,), dtype=_f32).at[col_s].add(
            vals_s * _sc_gather(v, row_s), indices_are_sorted=True)
        nrm = jnp.linalg.norm(w_) + 1e-12
        v = w_ / nrm
    A_rho = jax.lax.stop_gradient(jnp.maximum(nrm, 1e-6))
    k = KAPPA / A_rho
    u_all = _tc_sortdesc(W.T).T
    Wp = jax.vmap(lambda r, u: _l1_row_proj(r, u, k))(W, u_all)
    reg_loss = REG_COEF * jnp.sum(jax.nn.relu(jnp.sum(jnp.abs(Wp), axis=1) * A_rho - KAPPA))

    ir = (1.0 / jnp.sqrt(jnp.maximum(deg_r, 1.0)))[:, None]
    ic = (1.0 / jnp.sqrt(jnp.maximum(deg_c, 1.0)))[:, None]

    om1t = Omega_1.T
    wpt = Wp.T
    hwt = head_w.T
    hb = head_b.reshape(1, OUT)

    rowp, colp = _prep_edge_chunks(row, col)
    bt, z3 = _tc_init(x, om1t, ir)
    for _ in range(FW_ITERS - 2):
        xa = _sc_spmm_call(z3.reshape(2 * N, H), rowp, colp)
        z3 = _tc_mid(xa, ic, ir, wpt, bt)
    xa = _sc_spmm_call(z3.reshape(2 * N, H), rowp, colp)
    logits = _tc_final(xa, ic, wpt, bt, hwt, hb)
    return (logits, reg_loss)
